# baseline (speedup 1.0000x reference)
import functools

import jax
import jax.numpy as jnp
from jax import lax
from jax.experimental import pallas as pl
from jax.experimental.pallas import tpu as pltpu
from jax.experimental.pallas import tpu_sc as plsc

B, C, N, S, D = 8, 3, 8192, 2048, 256
ST = 1024
K = 3


def _lex_lt(va, ia, vb, ib):
    return (va < vb) | ((va == vb) & (ia < ib))


def _merge3(A, B):
    (av1, ai1, av2, ai2, av3, ai3) = A
    (bv1, bi1, bv2, bi2, bv3, bi3) = B
    t1 = _lex_lt(av1, ai1, bv1, bi1)
    m1v = jnp.where(t1, av1, bv1)
    m1i = jnp.where(t1, ai1, bi1)
    cav = jnp.where(t1, av2, av1)
    cai = jnp.where(t1, ai2, ai1)
    cbv = jnp.where(t1, bv1, bv2)
    cbi = jnp.where(t1, bi1, bi2)
    nav = jnp.where(t1, av3, av2)
    nai = jnp.where(t1, ai3, ai2)
    nbv = jnp.where(t1, bv2, bv3)
    nbi = jnp.where(t1, bi2, bi3)
    t2 = _lex_lt(cav, cai, cbv, cbi)
    m2v = jnp.where(t2, cav, cbv)
    m2i = jnp.where(t2, cai, cbi)
    cav2 = jnp.where(t2, nav, cav)
    cai2 = jnp.where(t2, nai, cai)
    cbv2 = jnp.where(t2, cbv, nbv)
    cbi2 = jnp.where(t2, cbi, nbi)
    t3 = _lex_lt(cav2, cai2, cbv2, cbi2)
    m3v = jnp.where(t3, cav2, cbv2)
    m3i = jnp.where(t3, cai2, cbi2)
    return (m1v, m1i, m2v, m2i, m3v, m3i)


NSPLIT = 1
NW_ = N // NSPLIT


def _top3_body(x1_ref, x2t_ref, idx_ref, w_ref):
    b = pl.program_id(0)
    x1 = x1_ref[0]
    x2 = x2t_ref[0]

    n2 = jnp.sum(x2 * x2, axis=1, keepdims=True)
    iota = lax.broadcasted_iota(jnp.int32, (ST, NW_), 1).astype(jnp.float32)
    inf = jnp.float32(jnp.inf)
    bigf = jnp.float32(NW_)

    best = None
    for h in range(NSPLIT):
        x1h = x1[:, h * NW_:(h + 1) * NW_]
        mm = lax.dot_general(x2, x1h, (((1,), (0,)), ((), ())),
                             preferred_element_type=jnp.float32)
        d = -2.0 * mm
        d = d + n2
        sq = x1h * x1h
        n1 = (sq[0:1, :] + sq[1:2, :]) + sq[2:3, :]
        d = d + n1

        offs = jnp.float32(h * NW_)
        tri = []
        for k in range(K):
            m = jnp.min(d, axis=1, keepdims=True)
            af = jnp.min(jnp.where(d == m, iota, bigf), axis=1,
                         keepdims=True)
            tri.extend((m, af + offs))
            if k < K - 1:
                d = jnp.where(iota == af, inf, d)
        best = tuple(tri) if best is None else _merge3(best, tuple(tri))

    mins = [best[0], best[2], best[4]]
    amins = [best[1].astype(jnp.int32), best[3].astype(jnp.int32),
             best[5].astype(jnp.int32)]

    r = [1.0 / (m + 1e-08) for m in mins]
    norm = r[0] + r[1] + r[2]
    w = [rk / norm for rk in r]

    lane = lax.broadcasted_iota(jnp.int32, (ST, 8), 1)
    boffs = b * N
    iv = jnp.broadcast_to(amins[0] + boffs, (ST, 8))
    iv = jnp.where(lane == 1, jnp.broadcast_to(amins[1] + boffs, (ST, 8)), iv)
    iv = jnp.where(lane == 2, jnp.broadcast_to(amins[2] + boffs, (ST, 8)), iv)
    iv = jnp.where(lane >= 3, 0, iv)
    idx_ref[0] = iv

    wv = jnp.broadcast_to(w[0], (ST, 8))
    wv = jnp.where(lane == 1, jnp.broadcast_to(w[1], (ST, 8)), wv)
    wv = jnp.where(lane == 2, jnp.broadcast_to(w[2], (ST, 8)), wv)
    wv = jnp.where(lane >= 3, jnp.float32(0.0), wv)
    w_ref[0] = wv


def _top3(xyz1, xyz2t):
    nb = xyz1.shape[0]
    return pl.pallas_call(
        _top3_body,
        grid=(nb, S // ST),
        in_specs=[
            pl.BlockSpec((1, C, N), lambda b, s: (b, 0, 0)),
            pl.BlockSpec((1, ST, C), lambda b, s: (b, s, 0)),
        ],
        out_specs=[
            pl.BlockSpec((1, ST, 8), lambda b, s: (b, s, 0)),
            pl.BlockSpec((1, ST, 8), lambda b, s: (b, s, 0)),
        ],
        out_shape=[
            jax.ShapeDtypeStruct((nb, S, 8), jnp.int32),
            jax.ShapeDtypeStruct((nb, S, 8), jnp.float32),
        ],
    )(xyz1, xyz2t)


BS = B * S
NW = 32
QPW = BS // NW
QC = 32
RPC = QC * K
NCH = QPW // QC


def _sc_combine(table, idx, w):
    nq = idx.shape[0] // K
    qpw = nq // NW
    nch = qpw // QC

    def body(table_hbm, idx_hbm, w_hbm, out_hbm,
             idx_v, w_v, rows_v, out_v, sem0, sem1):
        wid = lax.axis_index("s") * 2 + lax.axis_index("c")
        q_base = wid * qpw
        r_base = q_base * K
        sems = (sem0, sem1)

        pltpu.sync_copy(idx_hbm.at[pl.ds(r_base, qpw * K)], idx_v)
        pltpu.sync_copy(w_hbm.at[pl.ds(r_base, qpw * K)],
                        w_v.at[pl.ds(0, qpw * K)])
        pltpu.async_copy(table_hbm.at[idx_v.at[pl.ds(0, RPC)]],
                         rows_v.at[0], sem0)

        def pair(ch, carry):
            for sub in range(2):
                ci = ch * 2 + sub
                p = sub

                @pl.when(ci + 1 < nch)
                def _():
                    pltpu.async_copy(
                        table_hbm.at[idx_v.at[pl.ds((ci + 1) * RPC, RPC)]],
                        rows_v.at[1 - p], sems[1 - p])

                pltpu.make_async_copy(table_hbm.at[pl.ds(0, RPC)],
                                      rows_v.at[p], sems[p]).wait()

                def qbody(q, c2):
                    wvec = w_v[pl.ds(ci * RPC + q * K, 16)]
                    w0 = wvec[0]
                    w1 = wvec[1]
                    w2 = wvec[2]
                    for dc in range(D // 16):
                        sl = pl.ds(dc * 16, 16)
                        acc = rows_v[p, q * K + 0, sl] * w0
                        acc = acc + rows_v[p, q * K + 1, sl] * w1
                        acc = acc + rows_v[p, q * K + 2, sl] * w2
                        out_v[q, sl] = acc
                    return c2

                lax.fori_loop(0, QC, qbody, 0)
                pltpu.sync_copy(out_v,
                                out_hbm.at[pl.ds(q_base + ci * QC, QC)])
            return carry

        lax.fori_loop(0, nch // 2, pair, 0)

    mesh = plsc.VectorSubcoreMesh(core_axis_name="c", subcore_axis_name="s")
    fn = functools.partial(
        pl.kernel,
        out_type=jax.ShapeDtypeStruct((nq, D), jnp.float32),
        mesh=mesh,
        scratch_types=[
            pltpu.VMEM((qpw * K,), jnp.int32),
            pltpu.VMEM((qpw * K + 16,), jnp.float32),
            pltpu.VMEM((2, RPC, D), jnp.float32),
            pltpu.VMEM((QC, D), jnp.float32),
            pltpu.SemaphoreType.DMA,
            pltpu.SemaphoreType.DMA,
        ],
    )(body)
    return fn(table, idx, w)


def kernel(xyz1, xyz2, points1):
    x2t = jnp.transpose(xyz2, (0, 2, 1))
    table = jnp.transpose(points1, (0, 2, 1)).reshape(B * N, D)
    idx8, w8 = _top3(xyz1, x2t)
    idx3 = idx8[:, :, :K].reshape(-1)
    w3 = w8[:, :, :K].reshape(-1)
    out_rows = _sc_combine(table, idx3, w3)
    return jnp.transpose(out_rows.reshape(B, S, D), (0, 2, 1))


# speedup vs baseline: 1.0050x; 1.0050x over previous
import functools

import jax
import jax.numpy as jnp
from jax import lax
from jax.experimental import pallas as pl
from jax.experimental.pallas import tpu as pltpu
from jax.experimental.pallas import tpu_sc as plsc

B, C, N, S, D = 8, 3, 8192, 2048, 256
ST = 512
K = 3


def _lex_lt(va, ia, vb, ib):
    return (va < vb) | ((va == vb) & (ia < ib))


def _merge3(A, B):
    (av1, ai1, av2, ai2, av3, ai3) = A
    (bv1, bi1, bv2, bi2, bv3, bi3) = B
    t1 = _lex_lt(av1, ai1, bv1, bi1)
    m1v = jnp.where(t1, av1, bv1)
    m1i = jnp.where(t1, ai1, bi1)
    cav = jnp.where(t1, av2, av1)
    cai = jnp.where(t1, ai2, ai1)
    cbv = jnp.where(t1, bv1, bv2)
    cbi = jnp.where(t1, bi1, bi2)
    nav = jnp.where(t1, av3, av2)
    nai = jnp.where(t1, ai3, ai2)
    nbv = jnp.where(t1, bv2, bv3)
    nbi = jnp.where(t1, bi2, bi3)
    t2 = _lex_lt(cav, cai, cbv, cbi)
    m2v = jnp.where(t2, cav, cbv)
    m2i = jnp.where(t2, cai, cbi)
    cav2 = jnp.where(t2, nav, cav)
    cai2 = jnp.where(t2, nai, cai)
    cbv2 = jnp.where(t2, cbv, nbv)
    cbi2 = jnp.where(t2, cbi, nbi)
    t3 = _lex_lt(cav2, cai2, cbv2, cbi2)
    m3v = jnp.where(t3, cav2, cbv2)
    m3i = jnp.where(t3, cai2, cbi2)
    return (m1v, m1i, m2v, m2i, m3v, m3i)


NSPLIT = 1
NW_ = N // NSPLIT


def _top3_body(x1_ref, x2t_ref, idx_ref, w_ref):
    b = pl.program_id(0)
    x1 = x1_ref[0]
    x2 = x2t_ref[0]

    n2 = jnp.sum(x2 * x2, axis=1, keepdims=True)
    iota = lax.broadcasted_iota(jnp.int32, (ST, NW_), 1).astype(jnp.float32)
    inf = jnp.float32(jnp.inf)
    bigf = jnp.float32(NW_)

    best = None
    for h in range(NSPLIT):
        x1h = x1[:, h * NW_:(h + 1) * NW_]
        mm = lax.dot_general(x2, x1h, (((1,), (0,)), ((), ())),
                             preferred_element_type=jnp.float32)
        d = -2.0 * mm
        d = d + n2
        sq = x1h * x1h
        n1 = (sq[0:1, :] + sq[1:2, :]) + sq[2:3, :]
        d = d + n1

        offs = jnp.float32(h * NW_)
        tri = []
        for k in range(K):
            m = jnp.min(d, axis=1, keepdims=True)
            af = jnp.min(jnp.where(d == m, iota, bigf), axis=1,
                         keepdims=True)
            tri.extend((m, af + offs))
            if k < K - 1:
                d = jnp.where(iota == af, inf, d)
        best = tuple(tri) if best is None else _merge3(best, tuple(tri))

    mins = [best[0], best[2], best[4]]
    amins = [best[1].astype(jnp.int32), best[3].astype(jnp.int32),
             best[5].astype(jnp.int32)]

    r = [1.0 / (m + 1e-08) for m in mins]
    norm = r[0] + r[1] + r[2]
    w = [rk / norm for rk in r]

    lane = lax.broadcasted_iota(jnp.int32, (ST, 8), 1)
    boffs = b * N
    iv = jnp.broadcast_to(amins[0] + boffs, (ST, 8))
    iv = jnp.where(lane == 1, jnp.broadcast_to(amins[1] + boffs, (ST, 8)), iv)
    iv = jnp.where(lane == 2, jnp.broadcast_to(amins[2] + boffs, (ST, 8)), iv)
    iv = jnp.where(lane >= 3, 0, iv)
    idx_ref[0] = iv

    wv = jnp.broadcast_to(w[0], (ST, 8))
    wv = jnp.where(lane == 1, jnp.broadcast_to(w[1], (ST, 8)), wv)
    wv = jnp.where(lane == 2, jnp.broadcast_to(w[2], (ST, 8)), wv)
    wv = jnp.where(lane >= 3, jnp.float32(0.0), wv)
    w_ref[0] = wv


def _top3(xyz1, xyz2t):
    nb = xyz1.shape[0]
    return pl.pallas_call(
        _top3_body,
        grid=(nb, S // ST),
        in_specs=[
            pl.BlockSpec((1, C, N), lambda b, s: (b, 0, 0)),
            pl.BlockSpec((1, ST, C), lambda b, s: (b, s, 0)),
        ],
        out_specs=[
            pl.BlockSpec((1, ST, 8), lambda b, s: (b, s, 0)),
            pl.BlockSpec((1, ST, 8), lambda b, s: (b, s, 0)),
        ],
        out_shape=[
            jax.ShapeDtypeStruct((nb, S, 8), jnp.int32),
            jax.ShapeDtypeStruct((nb, S, 8), jnp.float32),
        ],
    )(xyz1, xyz2t)


BS = B * S
NW = 32
QPW = BS // NW
QC = 32
RPC = QC * K
NCH = QPW // QC


def _sc_combine(table, idx, w):
    nq = idx.shape[0] // K
    qpw = nq // NW
    nch = qpw // QC

    def body(table_hbm, idx_hbm, w_hbm, out_hbm,
             idx_v, w_v, rows_v, out_v, sem0, sem1):
        wid = lax.axis_index("s") * 2 + lax.axis_index("c")
        q_base = wid * qpw
        r_base = q_base * K
        sems = (sem0, sem1)

        pltpu.sync_copy(idx_hbm.at[pl.ds(r_base, qpw * K)], idx_v)
        pltpu.sync_copy(w_hbm.at[pl.ds(r_base, qpw * K)],
                        w_v.at[pl.ds(0, qpw * K)])
        pltpu.async_copy(table_hbm.at[idx_v.at[pl.ds(0, RPC)]],
                         rows_v.at[0], sem0)

        def pair(ch, carry):
            for sub in range(2):
                ci = ch * 2 + sub
                p = sub

                @pl.when(ci + 1 < nch)
                def _():
                    pltpu.async_copy(
                        table_hbm.at[idx_v.at[pl.ds((ci + 1) * RPC, RPC)]],
                        rows_v.at[1 - p], sems[1 - p])

                pltpu.make_async_copy(table_hbm.at[pl.ds(0, RPC)],
                                      rows_v.at[p], sems[p]).wait()

                def qbody(q, c2):
                    wvec = w_v[pl.ds(ci * RPC + q * K, 16)]
                    w0 = wvec[0]
                    w1 = wvec[1]
                    w2 = wvec[2]
                    for dc in range(D // 16):
                        sl = pl.ds(dc * 16, 16)
                        acc = rows_v[p, q * K + 0, sl] * w0
                        acc = acc + rows_v[p, q * K + 1, sl] * w1
                        acc = acc + rows_v[p, q * K + 2, sl] * w2
                        out_v[q, sl] = acc
                    return c2

                lax.fori_loop(0, QC, qbody, 0)
                pltpu.sync_copy(out_v,
                                out_hbm.at[pl.ds(q_base + ci * QC, QC)])
            return carry

        lax.fori_loop(0, nch // 2, pair, 0)

    mesh = plsc.VectorSubcoreMesh(core_axis_name="c", subcore_axis_name="s")
    fn = functools.partial(
        pl.kernel,
        out_type=jax.ShapeDtypeStruct((nq, D), jnp.float32),
        mesh=mesh,
        scratch_types=[
            pltpu.VMEM((qpw * K,), jnp.int32),
            pltpu.VMEM((qpw * K + 16,), jnp.float32),
            pltpu.VMEM((2, RPC, D), jnp.float32),
            pltpu.VMEM((QC, D), jnp.float32),
            pltpu.SemaphoreType.DMA,
            pltpu.SemaphoreType.DMA,
        ],
    )(body)
    return fn(table, idx, w)


def kernel(xyz1, xyz2, points1):
    x2t = jnp.transpose(xyz2, (0, 2, 1))
    table = jnp.transpose(points1, (0, 2, 1)).reshape(B * N, D)
    idx8, w8 = _top3(xyz1, x2t)
    idx3 = idx8[:, :, :K].reshape(-1)
    w3 = w8[:, :, :K].reshape(-1)
    out_rows = _sc_combine(table, idx3, w3)
    return jnp.transpose(out_rows.reshape(B, S, D), (0, 2, 1))


# speedup vs baseline: 1.0152x; 1.0101x over previous
import functools

import jax
import jax.numpy as jnp
from jax import lax
from jax.experimental import pallas as pl
from jax.experimental.pallas import tpu as pltpu
from jax.experimental.pallas import tpu_sc as plsc

B, C, N, S, D = 8, 3, 8192, 2048, 256
ST = 512
K = 3


def _lex_lt(va, ia, vb, ib):
    return (va < vb) | ((va == vb) & (ia < ib))


def _merge3(A, B):
    (av1, ai1, av2, ai2, av3, ai3) = A
    (bv1, bi1, bv2, bi2, bv3, bi3) = B
    t1 = _lex_lt(av1, ai1, bv1, bi1)
    m1v = jnp.where(t1, av1, bv1)
    m1i = jnp.where(t1, ai1, bi1)
    cav = jnp.where(t1, av2, av1)
    cai = jnp.where(t1, ai2, ai1)
    cbv = jnp.where(t1, bv1, bv2)
    cbi = jnp.where(t1, bi1, bi2)
    nav = jnp.where(t1, av3, av2)
    nai = jnp.where(t1, ai3, ai2)
    nbv = jnp.where(t1, bv2, bv3)
    nbi = jnp.where(t1, bi2, bi3)
    t2 = _lex_lt(cav, cai, cbv, cbi)
    m2v = jnp.where(t2, cav, cbv)
    m2i = jnp.where(t2, cai, cbi)
    cav2 = jnp.where(t2, nav, cav)
    cai2 = jnp.where(t2, nai, cai)
    cbv2 = jnp.where(t2, cbv, nbv)
    cbi2 = jnp.where(t2, cbi, nbi)
    t3 = _lex_lt(cav2, cai2, cbv2, cbi2)
    m3v = jnp.where(t3, cav2, cbv2)
    m3i = jnp.where(t3, cai2, cbi2)
    return (m1v, m1i, m2v, m2i, m3v, m3i)


NSPLIT = 1
NW_ = N // NSPLIT


def _top3_body(x1_ref, x2t_ref, idx_ref, w_ref):
    b = pl.program_id(0)
    x1 = x1_ref[0]
    x2 = x2t_ref[0]

    n2 = jnp.sum(x2 * x2, axis=1, keepdims=True)
    iota = lax.broadcasted_iota(jnp.int32, (ST, NW_), 1).astype(jnp.float32)
    inf = jnp.float32(jnp.inf)
    bigf = jnp.float32(NW_)

    best = None
    for h in range(NSPLIT):
        x1h = x1[:, h * NW_:(h + 1) * NW_]
        mm = lax.dot_general(x2, x1h, (((1,), (0,)), ((), ())),
                             preferred_element_type=jnp.float32)
        d = -2.0 * mm
        d = d + n2
        sq = x1h * x1h
        n1 = (sq[0:1, :] + sq[1:2, :]) + sq[2:3, :]
        d = d + n1

        offs = jnp.float32(h * NW_)
        tri = []
        for k in range(K):
            m = jnp.min(d, axis=1, keepdims=True)
            af = jnp.min(jnp.where(d == m, iota, bigf), axis=1,
                         keepdims=True)
            tri.extend((m, af + offs))
            if k < K - 1:
                d = jnp.where(iota == af, inf, d)
        best = tuple(tri) if best is None else _merge3(best, tuple(tri))

    mins = [best[0], best[2], best[4]]
    amins = [best[1].astype(jnp.int32), best[3].astype(jnp.int32),
             best[5].astype(jnp.int32)]

    r = [1.0 / (m + 1e-08) for m in mins]
    norm = r[0] + r[1] + r[2]
    w = [rk / norm for rk in r]

    lane = lax.broadcasted_iota(jnp.int32, (ST, 8), 1)
    boffs = b * N
    iv = jnp.broadcast_to(amins[0] + boffs, (ST, 8))
    iv = jnp.where(lane == 1, jnp.broadcast_to(amins[1] + boffs, (ST, 8)), iv)
    iv = jnp.where(lane == 2, jnp.broadcast_to(amins[2] + boffs, (ST, 8)), iv)
    iv = jnp.where(lane >= 3, 0, iv)
    idx_ref[0] = iv

    wv = jnp.broadcast_to(w[0], (ST, 8))
    wv = jnp.where(lane == 1, jnp.broadcast_to(w[1], (ST, 8)), wv)
    wv = jnp.where(lane == 2, jnp.broadcast_to(w[2], (ST, 8)), wv)
    wv = jnp.where(lane >= 3, jnp.float32(0.0), wv)
    w_ref[0] = wv


def _top3(xyz1, xyz2t):
    nb = xyz1.shape[0]
    return pl.pallas_call(
        _top3_body,
        grid=(nb, S // ST),
        in_specs=[
            pl.BlockSpec((1, C, N), lambda b, s: (b, 0, 0)),
            pl.BlockSpec((1, ST, C), lambda b, s: (b, s, 0)),
        ],
        out_specs=[
            pl.BlockSpec((1, ST, 8), lambda b, s: (b, s, 0)),
            pl.BlockSpec((1, ST, 8), lambda b, s: (b, s, 0)),
        ],
        out_shape=[
            jax.ShapeDtypeStruct((nb, S, 8), jnp.int32),
            jax.ShapeDtypeStruct((nb, S, 8), jnp.float32),
        ],
    )(xyz1, xyz2t)


BS = B * S
NW = 32
QPW = BS // NW
QC = 32
RPC = QC * K
NCH = QPW // QC


def _sc_combine(table, idx, w):
    nq = idx.shape[0] // K
    qpw = nq // NW
    nch = qpw // QC

    def body(table_hbm, idx_hbm, w_hbm, out_hbm,
             idx_v, w_v, rows_v, out_v, sem0, sem1, semo0, semo1):
        wid = lax.axis_index("s") * 2 + lax.axis_index("c")
        q_base = wid * qpw
        r_base = q_base * K
        sems = (sem0, sem1)
        semos = (semo0, semo1)

        pltpu.sync_copy(idx_hbm.at[pl.ds(r_base, qpw * K)], idx_v)
        pltpu.sync_copy(w_hbm.at[pl.ds(r_base, qpw * K)],
                        w_v.at[pl.ds(0, qpw * K)])
        pltpu.async_copy(table_hbm.at[idx_v.at[pl.ds(0, RPC)]],
                         rows_v.at[0], sem0)

        def pair(ch, carry):
            for sub in range(2):
                ci = ch * 2 + sub
                p = sub

                @pl.when(ci + 1 < nch)
                def _():
                    pltpu.async_copy(
                        table_hbm.at[idx_v.at[pl.ds((ci + 1) * RPC, RPC)]],
                        rows_v.at[1 - p], sems[1 - p])

                pltpu.make_async_copy(table_hbm.at[pl.ds(0, RPC)],
                                      rows_v.at[p], sems[p]).wait()

                @pl.when(ch > 0)
                def _():
                    pltpu.make_async_copy(
                        out_v.at[p], out_hbm.at[pl.ds(0, QC)],
                        semos[p]).wait()

                def qbody(q, c2):
                    wvec = w_v[pl.ds(ci * RPC + q * K, 16)]
                    w0 = wvec[0]
                    w1 = wvec[1]
                    w2 = wvec[2]
                    for dc in range(D // 16):
                        sl = pl.ds(dc * 16, 16)
                        acc = rows_v[p, q * K + 0, sl] * w0
                        acc = acc + rows_v[p, q * K + 1, sl] * w1
                        acc = acc + rows_v[p, q * K + 2, sl] * w2
                        out_v[p, q, sl] = acc
                    return c2

                lax.fori_loop(0, QC, qbody, 0)
                pltpu.async_copy(out_v.at[p],
                                 out_hbm.at[pl.ds(q_base + ci * QC, QC)],
                                 semos[p])
            return carry

        lax.fori_loop(0, nch // 2, pair, 0)
        for p in range(2):
            pltpu.make_async_copy(out_v.at[p], out_hbm.at[pl.ds(0, QC)],
                                  semos[p]).wait()

    mesh = plsc.VectorSubcoreMesh(core_axis_name="c", subcore_axis_name="s")
    fn = functools.partial(
        pl.kernel,
        out_type=jax.ShapeDtypeStruct((nq, D), jnp.float32),
        mesh=mesh,
        scratch_types=[
            pltpu.VMEM((qpw * K,), jnp.int32),
            pltpu.VMEM((qpw * K + 16,), jnp.float32),
            pltpu.VMEM((2, RPC, D), jnp.float32),
            pltpu.VMEM((2, QC, D), jnp.float32),
            pltpu.SemaphoreType.DMA,
            pltpu.SemaphoreType.DMA,
            pltpu.SemaphoreType.DMA,
            pltpu.SemaphoreType.DMA,
        ],
    )(body)
    return fn(table, idx, w)


def kernel(xyz1, xyz2, points1):
    x2t = jnp.transpose(xyz2, (0, 2, 1))
    table = jnp.transpose(points1, (0, 2, 1)).reshape(B * N, D)
    idx8, w8 = _top3(xyz1, x2t)
    idx3 = idx8[:, :, :K].reshape(-1)
    w3 = w8[:, :, :K].reshape(-1)
    out_rows = _sc_combine(table, idx3, w3)
    return jnp.transpose(out_rows.reshape(B, S, D), (0, 2, 1))
